# baseline (speedup 1.0000x reference)
import functools

import jax
import jax.numpy as jnp
from jax import lax
from jax.experimental import pallas as pl
from jax.experimental.pallas import tpu as pltpu

_BM = 400
_BN = 80


def _linear_kernel(x_ref, w_ref, o_ref):
    o_ref[...] = lax.dot_general(
        x_ref[...], w_ref[...], (((1,), (1,)), ((), ())),
        preferred_element_type=jnp.float32)


def _fused_kernel(n1, a_ref, u_ref, w2_ref, o_ref, v_ref, z_ref):
    s = pl.program_id(0)

    @pl.when(s < n1)
    def _phase1():
        h = lax.dot_general(
            a_ref[...], u_ref[...], (((1,), (0,)), ((), ())),
            preferred_element_type=jnp.float32)
        h = jnp.maximum(h, 0.0)
        v_ref[pl.ds(s * _BM, _BM), :] = lax.dot_general(
            h, w2_ref[...], (((1,), (1,)), ((), ())),
            preferred_element_type=jnp.float32)

    @pl.when((s >= n1) & (s < 2 * n1))
    def _phase2():
        z_ref[pl.ds((s - n1) * _BM, _BM), :] = lax.dot_general(
            a_ref[...], v_ref[...], (((1,), (0,)), ((), ())),
            preferred_element_type=jnp.float32)

    @pl.when(s >= 2 * n1)
    def _phase3():
        zi = z_ref[pl.ds((s - 2 * n1) * _BN, _BN), :]
        logits = lax.dot_general(
            zi, z_ref[...], (((1,), (1,)), ((), ())),
            preferred_element_type=jnp.float32)
        o_ref[...] = jax.nn.sigmoid(logits)


def kernel(X, A_tilde, W1, W2):
    N, _ = X.shape
    H = W1.shape[0]
    L = W2.shape[0]

    U = pl.pallas_call(
        _linear_kernel,
        out_shape=jax.ShapeDtypeStruct((N, H), jnp.float32),
    )(X, W1)

    n1 = N // _BM
    n3 = N // _BN
    num_steps = 2 * n1 + n3

    def a_map(s):
        return (jnp.minimum(jnp.where(s < n1, s, s - n1), n1 - 1), 0)

    def o_map(s):
        return (jnp.maximum(s - 2 * n1, 0), 0)

    A_hat = pl.pallas_call(
        functools.partial(_fused_kernel, n1),
        grid=(num_steps,),
        in_specs=[
            pl.BlockSpec((_BM, N), a_map),
            pl.BlockSpec((N, H), lambda s: (0, 0)),
            pl.BlockSpec((L, H), lambda s: (0, 0)),
        ],
        out_specs=pl.BlockSpec((_BN, N), o_map),
        out_shape=jax.ShapeDtypeStruct((N, N), jnp.float32),
        scratch_shapes=[
            pltpu.VMEM((N, L), jnp.float32),
            pltpu.VMEM((N, L), jnp.float32),
        ],
        compiler_params=pltpu.CompilerParams(
            dimension_semantics=("arbitrary",)),
    )(A_tilde, U, W2)

    return (A_hat, jnp.array([0]), jnp.array([0]))


# speedup vs baseline: 1.0812x; 1.0812x over previous
import functools

import jax
import jax.numpy as jnp
from jax import lax
from jax.experimental import pallas as pl
from jax.experimental.pallas import tpu as pltpu

_BM = 200
_BN = 200


def _linear_kernel(x_ref, w_ref, o_ref):
    o_ref[...] = lax.dot_general(
        x_ref[...], w_ref[...], (((1,), (1,)), ((), ())),
        preferred_element_type=jnp.float32)


def _fused_kernel(n1, a_ref, u_ref, w2_ref, o_ref, v_ref, z_ref):
    s = pl.program_id(0)

    @pl.when(s < n1)
    def _phase1():
        h = lax.dot_general(
            a_ref[...], u_ref[...], (((1,), (0,)), ((), ())),
            preferred_element_type=jnp.float32)
        h = jnp.maximum(h, 0.0)
        v_ref[pl.ds(s * _BM, _BM), :] = lax.dot_general(
            h, w2_ref[...], (((1,), (1,)), ((), ())),
            preferred_element_type=jnp.float32)

    @pl.when((s >= n1) & (s < 2 * n1))
    def _phase2():
        z_ref[pl.ds((s - n1) * _BM, _BM), :] = lax.dot_general(
            a_ref[...], v_ref[...], (((1,), (0,)), ((), ())),
            preferred_element_type=jnp.float32)

    @pl.when(s >= 2 * n1)
    def _phase3():
        zi = z_ref[pl.ds((s - 2 * n1) * _BN, _BN), :]
        logits = lax.dot_general(
            zi, z_ref[...], (((1,), (1,)), ((), ())),
            preferred_element_type=jnp.float32)
        o_ref[...] = jax.nn.sigmoid(logits)


def kernel(X, A_tilde, W1, W2):
    N, _ = X.shape
    H = W1.shape[0]
    L = W2.shape[0]

    U = pl.pallas_call(
        _linear_kernel,
        out_shape=jax.ShapeDtypeStruct((N, H), jnp.float32),
    )(X, W1)

    n1 = N // _BM
    n3 = N // _BN
    num_steps = 2 * n1 + n3

    def a_map(s):
        return (jnp.minimum(jnp.where(s < n1, s, s - n1), n1 - 1), 0)

    def o_map(s):
        return (jnp.maximum(s - 2 * n1, 0), 0)

    A_hat = pl.pallas_call(
        functools.partial(_fused_kernel, n1),
        grid=(num_steps,),
        in_specs=[
            pl.BlockSpec((_BM, N), a_map),
            pl.BlockSpec((N, H), lambda s: (0, 0)),
            pl.BlockSpec((L, H), lambda s: (0, 0)),
        ],
        out_specs=pl.BlockSpec((_BN, N), o_map),
        out_shape=jax.ShapeDtypeStruct((N, N), jnp.float32),
        scratch_shapes=[
            pltpu.VMEM((N, L), jnp.float32),
            pltpu.VMEM((N, L), jnp.float32),
        ],
        compiler_params=pltpu.CompilerParams(
            dimension_semantics=("arbitrary",)),
    )(A_tilde, U, W2)

    return (A_hat, jnp.array([0]), jnp.array([0]))


# speedup vs baseline: 1.1059x; 1.0229x over previous
import functools

import jax
import jax.numpy as jnp
from jax import lax
from jax.experimental import pallas as pl
from jax.experimental.pallas import tpu as pltpu

_BM = 400
_BN = 400


def _linear_kernel(x_ref, w_ref, o_ref):
    o_ref[...] = lax.dot_general(
        x_ref[...], w_ref[...], (((1,), (1,)), ((), ())),
        preferred_element_type=jnp.float32)


def _prop_kernel(n1, a_ref, u_ref, w2_ref, z_ref, v_ref):
    s = pl.program_id(0)

    @pl.when(s < n1)
    def _phase1():
        h = lax.dot_general(
            a_ref[...], u_ref[...], (((1,), (0,)), ((), ())),
            preferred_element_type=jnp.float32)
        h = jnp.maximum(h, 0.0)
        v_ref[pl.ds(s * _BM, _BM), :] = lax.dot_general(
            h, w2_ref[...], (((1,), (1,)), ((), ())),
            preferred_element_type=jnp.float32)

    @pl.when(s >= n1)
    def _phase2():
        z_ref[...] = lax.dot_general(
            a_ref[...], v_ref[...], (((1,), (0,)), ((), ())),
            preferred_element_type=jnp.float32)


def _decoder_kernel(zi_ref, z_ref, o_ref):
    logits = lax.dot_general(
        zi_ref[...], z_ref[...], (((1,), (1,)), ((), ())),
        preferred_element_type=jnp.float32)
    o_ref[...] = jax.nn.sigmoid(logits)


def kernel(X, A_tilde, W1, W2):
    N, _ = X.shape
    H = W1.shape[0]
    L = W2.shape[0]

    U = pl.pallas_call(
        _linear_kernel,
        out_shape=jax.ShapeDtypeStruct((N, H), jnp.float32),
    )(X, W1)

    n1 = N // _BM

    def a_map(s):
        return (jnp.where(s < n1, s, s - n1), 0)

    def z_map(s):
        return (jnp.maximum(s - n1, 0), 0)

    Z = pl.pallas_call(
        functools.partial(_prop_kernel, n1),
        grid=(2 * n1,),
        in_specs=[
            pl.BlockSpec((_BM, N), a_map),
            pl.BlockSpec((N, H), lambda s: (0, 0)),
            pl.BlockSpec((L, H), lambda s: (0, 0)),
        ],
        out_specs=pl.BlockSpec((_BM, L), z_map),
        out_shape=jax.ShapeDtypeStruct((N, L), jnp.float32),
        scratch_shapes=[
            pltpu.VMEM((N, L), jnp.float32),
        ],
        compiler_params=pltpu.CompilerParams(
            dimension_semantics=("arbitrary",)),
    )(A_tilde, U, W2)

    A_hat = pl.pallas_call(
        _decoder_kernel,
        grid=(N // _BN,),
        in_specs=[
            pl.BlockSpec((_BN, L), lambda i: (i, 0)),
            pl.BlockSpec((N, L), lambda i: (0, 0)),
        ],
        out_specs=pl.BlockSpec((_BN, N), lambda i: (i, 0)),
        out_shape=jax.ShapeDtypeStruct((N, N), jnp.float32),
        compiler_params=pltpu.CompilerParams(
            dimension_semantics=("parallel",)),
    )(Z, Z)

    return (A_hat, jnp.array([0]), jnp.array([0]))
